# baseline (speedup 1.0000x reference)
import functools

import jax
import jax.numpy as jnp
from jax import lax
from jax.experimental import pallas as pl
from jax.experimental.pallas import tpu as pltpu
from jax.experimental.pallas import tpu_sc as plsc

_N, _E, _D, _ED = 10000, 160000, 256, 16
_GD = _D
_DH = _D // 2
_NSUB = 16
_NW = 2 * _NSUB
_GW = 40
_SW = 80
_KC = 5
_CH = _E // _KC

_f32 = jnp.float32


def _sigmoid(z):
    return jax.nn.sigmoid(z)


def _silu(z):
    return z * _sigmoid(z)


def _pre_body(h_ref, x16_ref, w1a_ref, w1b_ref, be1_ref, hax_ref, hbx_ref):
    h = h_ref[...]
    u32 = jnp.uint32
    bf16 = jnp.bfloat16
    xe = jnp.concatenate(
        [x16_ref[...], jnp.zeros((h.shape[0], _D - 16), _f32)], axis=1)
    pe = lax.bitcast_convert_type(xe.astype(bf16).astype(_f32), u32)
    ha = jnp.dot(h, w1a_ref[...], preferred_element_type=_f32) + be1_ref[...]
    hb = jnp.dot(h, w1b_ref[...], preferred_element_type=_f32)
    pa = lax.bitcast_convert_type(ha.astype(bf16).astype(_f32), u32)
    pb = lax.bitcast_convert_type(hb.astype(bf16).astype(_f32), u32)
    hax_ref[...] = lax.bitcast_convert_type((pa >> 16) | pe, _f32)
    hbx_ref[...] = lax.bitcast_convert_type((pb >> 16) | pe, _f32)


def _tc_pre(h, x16, w1a, w1b, be1):
    bn = 2000
    return pl.pallas_call(
        _pre_body,
        grid=(_N // bn,),
        in_specs=[
            pl.BlockSpec((bn, _D), lambda i: (i, 0)),
            pl.BlockSpec((bn, 16), lambda i: (i, 0)),
            pl.BlockSpec((_D, _D), lambda i: (0, 0)),
            pl.BlockSpec((_D, _D), lambda i: (0, 0)),
            pl.BlockSpec((1, _D), lambda i: (0, 0)),
        ],
        out_specs=[
            pl.BlockSpec((bn, _GD), lambda i: (i, 0)),
            pl.BlockSpec((bn, _GD), lambda i: (i, 0)),
        ],
        out_shape=[jax.ShapeDtypeStruct((_N, _GD), _f32)] * 2,
    )(h, x16, w1a, w1b, be1)


def _sc_gather(hax, hbx, row, col, k):
    ewc = _CH // _NW
    nwin = ewc // _GW
    mesh = plsc.VectorSubcoreMesh(core_axis_name="c", subcore_axis_name="s")

    @functools.partial(
        pl.kernel,
        out_type=(
            jax.ShapeDtypeStruct((_CH, _GD), _f32),
            jax.ShapeDtypeStruct((_CH, _GD), _f32),
        ),
        mesh=mesh,
        scratch_types=[
            pltpu.VMEM((ewc,), jnp.int32),
            pltpu.VMEM((ewc,), jnp.int32),
            pltpu.VMEM((_GW, _GD), _f32),
            pltpu.VMEM((_GW, _GD), _f32),
            pltpu.VMEM((_GW, _GD), _f32),
            pltpu.VMEM((_GW, _GD), _f32),
            pltpu.VMEM((_GW, _GD), _f32),
            pltpu.VMEM((_GW, _GD), _f32),
            pltpu.SemaphoreType.DMA,
            pltpu.SemaphoreType.DMA,
            pltpu.SemaphoreType.DMA,
            pltpu.SemaphoreType.DMA,
            pltpu.SemaphoreType.DMA,
            pltpu.SemaphoreType.DMA,
            pltpu.SemaphoreType.DMA,
        ],
    )
    def gk(hax_hbm, hbx_hbm, row_hbm, col_hbm,
           har_o, hbc_o,
           idxr, idxc, ba0, ba1, ba2, bb0, bb1, bb2,
           gs0, gs1, gs2, ws0, ws1, ws2, isem):
        wid = lax.axis_index("s") * 2 + lax.axis_index("c")
        gbase = _CH * k + wid * ewc
        lbase = wid * ewc
        pltpu.async_copy(row_hbm.at[pl.ds(gbase, ewc)], idxr, isem).wait()
        pltpu.async_copy(col_hbm.at[pl.ds(gbase, ewc)], idxc, isem).wait()

        bufa = (ba0, ba1, ba2)
        bufb = (bb0, bb1, bb2)
        gsem = (gs0, gs1, gs2)
        wsem = (ws0, ws1, ws2)

        def issue_gather(w, p):
            sl = pl.ds(w * _GW, _GW)
            pltpu.async_copy(hax_hbm.at[idxr.at[sl]], bufa[p], gsem[p])
            pltpu.async_copy(hbx_hbm.at[idxc.at[sl]], bufb[p], gsem[p])

        def wait_gather(p):
            sl = pl.ds(0, _GW)
            pltpu.make_async_copy(hax_hbm.at[idxr.at[sl]], bufa[p], gsem[p]).wait()
            pltpu.make_async_copy(hbx_hbm.at[idxc.at[sl]], bufb[p], gsem[p]).wait()

        def issue_write(w, p):
            off = lbase + w * _GW
            pltpu.async_copy(bufa[p], har_o.at[pl.ds(off, _GW)], wsem[p])
            pltpu.async_copy(bufb[p], hbc_o.at[pl.ds(off, _GW)], wsem[p])

        def wait_write(p):
            sl = pl.ds(lbase, _GW)
            pltpu.make_async_copy(bufa[p], har_o.at[sl], wsem[p]).wait()
            pltpu.make_async_copy(bufb[p], hbc_o.at[sl], wsem[p]).wait()

        issue_gather(0, 0)
        issue_gather(1, 1)
        wait_gather(0)
        issue_write(0, 0)
        issue_gather(2, 2)
        wait_gather(1)
        issue_write(1, 1)
        wait_write(0)
        issue_gather(3, 0)

        @pl.loop(2, nwin - 3, step=3)
        def _(t):
            for j in range(3):
                p = (2 + j) % 3
                q = (p + 2) % 3
                wait_gather(p)
                issue_write(t + j, p)
                wait_write(q)
                issue_gather(t + j + 2, q)

        wait_gather(2)
        issue_write(nwin - 2, 2)
        wait_gather(0)
        issue_write(nwin - 1, 0)
        wait_write(0)
        wait_write(1)
        wait_write(2)

    return gk(hax, hbx, row, col)


def _edge_body(harx_ref, hbcx_ref, ea_ref,
               w1c_ref, w1d_ref, we2_ref, be2_ref, wa_ref, ba_ref,
               wc1_ref, bc1_ref, wc2_ref,
               wm0_ref, wm1_ref, cu_ref):
    u32 = jnp.uint32
    wa_u = lax.bitcast_convert_type(harx_ref[...], u32)
    wb_u = lax.bitcast_convert_type(hbcx_ref[...], u32)
    har = lax.bitcast_convert_type(wa_u << 16, _f32)
    hbc = lax.bitcast_convert_type(wb_u << 16, _f32)
    himask = u32(0xFFFF0000)
    xr = lax.bitcast_convert_type(wa_u & himask, _f32)[:, :16]
    xc = lax.bitcast_convert_type(wb_u & himask, _f32)[:, :16]
    rel = xr - xc
    d2 = jnp.sum(rel * rel, axis=1, keepdims=True)
    dist = jnp.sqrt(d2)
    pre1 = (har + hbc + dist * w1c_ref[...]
            + lax.dot_general(ea_ref[...], w1d_ref[...],
                              (((0,), (0,)), ((), ())),
                              preferred_element_type=_f32))
    e1 = _silu(pre1)
    pre2 = jnp.dot(e1, we2_ref[...],
                   preferred_element_type=_f32) + be2_ref[...]
    m = _silu(pre2)
    att = _sigmoid(
        jnp.sum(m * wa_ref[...], axis=1, keepdims=True) + ba_ref[...])
    wm = att * m
    wm0_ref[...] = wm[:, :_DH]
    wm1_ref[...] = wm[:, _DH:]
    t = _silu(jnp.dot(m, wc1_ref[...],
                      preferred_element_type=_f32) + bc1_ref[...])
    cw = jnp.sum(t * wc2_ref[...], axis=1, keepdims=True)
    cuv = (cw / (dist + 1e-8)) * rel
    cu_ref[...] = jnp.concatenate(
        [cuv, jnp.zeros((cuv.shape[0], 128 - 16), _f32)], axis=1)


def _tc_edge(harx, hbcx, ea, k, w1c, w1d, we2, be2, wa, ba, wc1, bc1, wc2):
    ne = harx.shape[0]
    be = 3200 if ne % 3200 == 0 else ne
    kb = k * (ne // be)
    full = lambda a, b: pl.BlockSpec((a, b), lambda i: (0, 0))
    return pl.pallas_call(
        _edge_body,
        grid=(ne // be,),
        in_specs=[
            pl.BlockSpec((be, _GD), lambda i: (i, 0)),
            pl.BlockSpec((be, _GD), lambda i: (i, 0)),
            pl.BlockSpec((_ED, be), lambda i: (0, i + kb)),
            full(1, _D),
            full(_ED, _D),
            full(_D, _D),
            full(1, _D),
            full(1, _D),
            full(1, 1),
            full(_D, _D),
            full(1, _D),
            full(1, _D),
        ],
        out_specs=[
            pl.BlockSpec((be, _DH), lambda i: (i, 0)),
            pl.BlockSpec((be, _DH), lambda i: (i, 0)),
            pl.BlockSpec((be, 128), lambda i: (i, 0)),
        ],
        out_shape=[
            jax.ShapeDtypeStruct((ne, _DH), _f32),
            jax.ShapeDtypeStruct((ne, _DH), _f32),
            jax.ShapeDtypeStruct((ne, 128), _f32),
        ],
    )(harx, hbcx, ea, w1c, w1d, we2, be2, wa, ba, wc1, bc1, wc2)


def _sc_scatter_msg(wm0s, wm1s, row, zeros_hbm):
    spc = _CH // _NSUB
    nwin = spc // _SW
    mesh = plsc.VectorSubcoreMesh(core_axis_name="c", subcore_axis_name="s")

    @functools.partial(
        pl.kernel,
        out_type=(
            jax.ShapeDtypeStruct((_N, _DH), _f32),
            jax.ShapeDtypeStruct((_N, _DH), _f32),
        ),
        mesh=mesh,
        scratch_types=[
            pltpu.VMEM((_SW,), jnp.int32),
            pltpu.VMEM((_SW,), jnp.int32),
            pltpu.VMEM((_SW, _DH), _f32),
            pltpu.VMEM((_SW, _DH), _f32),
            pltpu.VMEM_SHARED((_N, _DH), _f32),
            pltpu.SemaphoreType.DMA,
            pltpu.SemaphoreType.DMA,
        ],
    )
    def sk(*refs):
        wm0_hbm = refs[0:_KC]
        wm1_hbm = refs[_KC:2 * _KC]
        row_hbm, z_hbm, mi0_o, mi1_o = refs[2 * _KC:2 * _KC + 4]
        idxw0, idxw1, upd0, upd1, acc_sh, rs0, rs1 = refs[2 * _KC + 4:]
        c = lax.axis_index("c")
        s = lax.axis_index("s")
        @pl.when(s < 10)
        def _():
            pltpu.sync_copy(z_hbm, acc_sh.at[pl.ds(s * 1000, 1000)])

        plsc.subcore_barrier()

        idxw = (idxw0, idxw1)
        upd = (upd0, upd1)
        rsem = (rs0, rs1)

        for k in range(_KC):
            wm0k = wm0_hbm[k]
            wm1k = wm1_hbm[k]
            gbase = k * _CH + s * spc
            lbase = s * spc

            def issue_read(w, p):
                pltpu.async_copy(
                    row_hbm.at[pl.ds(gbase + w * _SW, _SW)], idxw[p], rsem[p])

                @pl.when(c == 0)
                def _():
                    pltpu.async_copy(
                        wm0k.at[pl.ds(lbase + w * _SW, _SW)], upd[p], rsem[p])

                @pl.when(c == 1)
                def _():
                    pltpu.async_copy(
                        wm1k.at[pl.ds(lbase + w * _SW, _SW)], upd[p], rsem[p])

            def wait_read(p):
                pltpu.make_async_copy(
                    row_hbm.at[pl.ds(gbase, _SW)], idxw[p], rsem[p]).wait()
                pltpu.make_async_copy(
                    wm0k.at[pl.ds(lbase, _SW)], upd[p], rsem[p]).wait()

            def scatter(p):
                pltpu.sync_copy(upd[p], acc_sh.at[idxw[p]], add=True)

            issue_read(0, 0)

            @pl.loop(0, nwin - 1, step=2)
            def _(t):
                issue_read(t + 1, 1)
                wait_read(0)
                scatter(0)
                issue_read(t + 2, 0)
                wait_read(1)
                scatter(1)

            wait_read(0)
            scatter(0)

        plsc.subcore_barrier()

        @pl.when(s < 10)
        def _():
            rows = pl.ds(s * 1000, 1000)

            @pl.when(c == 0)
            def _():
                pltpu.sync_copy(acc_sh.at[rows], mi0_o.at[rows])

            @pl.when(c == 1)
            def _():
                pltpu.sync_copy(acc_sh.at[rows], mi1_o.at[rows])

    return sk(*wm0s, *wm1s, row, zeros_hbm)


def _sc_scatter_coord(cus, row, zeros_hbm):
    ewc = _CH // _NW
    nwin = ewc // _GW
    mesh = plsc.VectorSubcoreMesh(core_axis_name="c", subcore_axis_name="s")

    @functools.partial(
        pl.kernel,
        out_type=(
            jax.ShapeDtypeStruct((_N, 128), _f32),
            jax.ShapeDtypeStruct((_N, 128), _f32),
        ),
        mesh=mesh,
        scratch_types=[
            pltpu.VMEM((_GW,), jnp.int32),
            pltpu.VMEM((_GW,), jnp.int32),
            pltpu.VMEM((_GW, 128), _f32),
            pltpu.VMEM((_GW, 128), _f32),
            pltpu.VMEM_SHARED((_N, 128), _f32),
            pltpu.SemaphoreType.DMA,
            pltpu.SemaphoreType.DMA,
        ],
    )
    def sk(*refs):
        cu_hbm = refs[0:_KC]
        row_hbm, z_hbm, cacc0_o, cacc1_o = refs[_KC:_KC + 4]
        idxw0, idxw1, cu0, cu1, cacc_sh, rs0, rs1 = refs[_KC + 4:]
        c = lax.axis_index("c")
        s = lax.axis_index("s")
        wid = s * 2 + c
        @pl.when(s < 10)
        def _():
            pltpu.sync_copy(z_hbm, cacc_sh.at[pl.ds(s * 1000, 1000)])

        plsc.subcore_barrier()

        idxw = (idxw0, idxw1)
        cub = (cu0, cu1)
        rsem = (rs0, rs1)

        for k in range(_KC):
            cuk = cu_hbm[k]
            gbase = k * _CH + wid * ewc
            lbase = wid * ewc

            def issue_read(w, p):
                pltpu.async_copy(
                    row_hbm.at[pl.ds(gbase + w * _GW, _GW)], idxw[p], rsem[p])
                pltpu.async_copy(
                    cuk.at[pl.ds(lbase + w * _GW, _GW)], cub[p], rsem[p])

            def wait_read(p):
                pltpu.make_async_copy(
                    row_hbm.at[pl.ds(gbase, _GW)], idxw[p], rsem[p]).wait()
                pltpu.make_async_copy(
                    cuk.at[pl.ds(lbase, _GW)], cub[p], rsem[p]).wait()

            def scatter(p):
                pltpu.sync_copy(cub[p], cacc_sh.at[idxw[p]], add=True)

            issue_read(0, 0)

            @pl.loop(0, nwin - 1, step=2)
            def _(t):
                issue_read(t + 1, 1)
                wait_read(0)
                scatter(0)
                issue_read(t + 2, 0)
                wait_read(1)
                scatter(1)

            wait_read(0)
            scatter(0)

        plsc.subcore_barrier()

        @pl.when(s < 10)
        def _():
            rows = pl.ds(s * 1000, 1000)

            @pl.when(c == 0)
            def _():
                pltpu.sync_copy(cacc_sh.at[rows], cacc0_o.at[rows])

            @pl.when(c == 1)
            def _():
                pltpu.sync_copy(cacc_sh.at[rows], cacc1_o.at[rows])

    return sk(*cus, row, zeros_hbm)


def _node_body(h_ref, m0_ref, m1_ref, x_ref, cacc0_ref, cacc1_ref,
               wn1a_ref, wn1b0_ref, wn1b1_ref, bn1_ref, wn2_ref, bn2_ref,
               hn_ref, xn_ref):
    pre = (jnp.dot(h_ref[...], wn1a_ref[...], preferred_element_type=_f32)
           + jnp.dot(m0_ref[...], wn1b0_ref[...], preferred_element_type=_f32)
           + jnp.dot(m1_ref[...], wn1b1_ref[...], preferred_element_type=_f32)
           + bn1_ref[...])
    u = _silu(pre)
    hn_ref[...] = (jnp.dot(u, wn2_ref[...], preferred_element_type=_f32)
                   + bn2_ref[...] + h_ref[...])
    xn_ref[...] = x_ref[...] + cacc0_ref[:, :16] + cacc1_ref[:, :16]


def _tc_node(h, m0, m1, x16, cacc0, cacc1, wn1a, wn1b0, wn1b1, bn1, wn2, bn2):
    bn = 2000
    full = lambda a, b: pl.BlockSpec((a, b), lambda i: (0, 0))
    return pl.pallas_call(
        _node_body,
        grid=(_N // bn,),
        in_specs=[
            pl.BlockSpec((bn, _D), lambda i: (i, 0)),
            pl.BlockSpec((bn, _DH), lambda i: (i, 0)),
            pl.BlockSpec((bn, _DH), lambda i: (i, 0)),
            pl.BlockSpec((bn, 16), lambda i: (i, 0)),
            pl.BlockSpec((bn, 128), lambda i: (i, 0)),
            pl.BlockSpec((bn, 128), lambda i: (i, 0)),
            full(_D, _D),
            full(_DH, _D),
            full(_DH, _D),
            full(1, _D),
            full(_D, _D),
            full(1, _D),
        ],
        out_specs=[
            pl.BlockSpec((bn, _D), lambda i: (i, 0)),
            pl.BlockSpec((bn, 16), lambda i: (i, 0)),
        ],
        out_shape=[
            jax.ShapeDtypeStruct((_N, _D), _f32),
            jax.ShapeDtypeStruct((_N, 16), _f32),
        ],
    )(h, m0, m1, x16, cacc0, cacc1, wn1a, wn1b0, wn1b1, bn1, wn2, bn2)


def kernel(h, x, edge_index, edge_attr,
           W_e1, b_e1, W_e2, b_e2,
           W_n1, b_n1, W_n2, b_n2,
           W_c1, b_c1, W_c2, W_a, b_a):
    row = edge_index[0]
    col = edge_index[1]
    w1a = W_e1[:_D]
    w1b = W_e1[_D:2 * _D]
    w1c = W_e1[2 * _D:2 * _D + 1]
    w1d = W_e1[2 * _D + 1:]
    x16 = jnp.pad(x, ((0, 0), (0, 16 - x.shape[1])))
    ea_t = edge_attr.T

    hax, hbx = _tc_pre(h, x16, w1a, w1b, b_e1.reshape(1, _D))

    wm0s, wm1s, cus = [], [], []
    for k in range(_KC):
        harx, hbcx = _sc_gather(hax, hbx, row, col, k)
        wm0, wm1, cu = _tc_edge(
            harx, hbcx, ea_t, k,
            w1c, w1d, W_e2, b_e2.reshape(1, _D),
            W_a.reshape(1, _D), b_a.reshape(1, 1),
            W_c1, b_c1.reshape(1, _D), W_c2.reshape(1, _D))
        wm0s.append(wm0)
        wm1s.append(wm1)
        cus.append(cu)

    zeros_hbm = jnp.zeros((1000, 128), _f32)
    mi0, mi1 = _sc_scatter_msg(wm0s, wm1s, row, zeros_hbm)
    cacc0, cacc1 = _sc_scatter_coord(cus, row, zeros_hbm)
    hn, xn16 = _tc_node(
        h, mi0, mi1, x16, cacc0, cacc1,
        W_n1[:_D], W_n1[_D:_D + _DH], W_n1[_D + _DH:],
        b_n1.reshape(1, _D), W_n2, b_n2.reshape(1, _D))
    return hn, xn16[:, :x.shape[1]]
